# baseline (speedup 1.0000x reference)
import dataclasses
import functools

import jax
import jax.numpy as jnp
from jax import lax
from jax.experimental import pallas as pl
from jax.experimental.pallas import tpu as pltpu
from jax.experimental.pallas import tpu_sc as plsc

B = 16384
D = 32
H = 128
NC = 2
NS = 16
NW = NC * NS
BPW = B // NW


def _sc_compiler_params():
    cp = pltpu.CompilerParams()
    if "needs_layout_passes" in pltpu.CompilerParams.__dataclass_fields__:
        cp = dataclasses.replace(cp, needs_layout_passes=False)
    return cp


def _gather_sc(user_table, item_table, user, item):
    mesh = plsc.VectorSubcoreMesh(core_axis_name="c", subcore_axis_name="s")

    @functools.partial(
        pl.kernel,
        mesh=mesh,
        out_type=[
            jax.ShapeDtypeStruct((B, D), jnp.float32),
            jax.ShapeDtypeStruct((B, D), jnp.float32),
        ],
        scratch_types=[
            pltpu.VMEM((BPW,), jnp.int32),
            pltpu.VMEM((BPW,), jnp.int32),
            pltpu.SemaphoreType.DMA,
        ],
        compiler_params=_sc_compiler_params(),
    )
    def k(ut_hbm, it_hbm, u_hbm, i_hbm, ue_hbm, ie_hbm, uv, iv, sem):
        wid = lax.axis_index("s") * NC + lax.axis_index("c")
        base = wid * BPW
        pltpu.sync_copy(u_hbm.at[pl.ds(base, BPW)], uv)
        pltpu.sync_copy(i_hbm.at[pl.ds(base, BPW)], iv)
        lanes = lax.iota(jnp.int32, 16)

        @pl.loop(0, BPW, step=16)
        def _(c):
            uchunk = uv[pl.ds(c, 16)]
            ichunk = iv[pl.ds(c, 16)]
            for j in range(16):
                uidx = jnp.sum(jnp.where(lanes == j, uchunk, 0))
                iidx = jnp.sum(jnp.where(lanes == j, ichunk, 0))
                pltpu.async_copy(ut_hbm.at[uidx], ue_hbm.at[base + c + j], sem)
                pltpu.async_copy(it_hbm.at[iidx], ie_hbm.at[base + c + j], sem)

        pltpu.make_async_copy(ut_hbm.at[pl.ds(0, BPW)],
                              ue_hbm.at[pl.ds(base, BPW)], sem).wait()
        pltpu.make_async_copy(it_hbm.at[pl.ds(0, BPW)],
                              ie_hbm.at[pl.ds(base, BPW)], sem).wait()

    return k(user_table, item_table, user, item)


def _mlp_body(ue_ref, ie_ref, w1u_ref, w1i_ref, b1_ref, w2_ref, b2_ref, o_ref):
    h = jnp.dot(ue_ref[...], w1u_ref[...], preferred_element_type=jnp.float32,
                precision=lax.Precision.HIGHEST)
    h = h + jnp.dot(ie_ref[...], w1i_ref[...], preferred_element_type=jnp.float32,
                    precision=lax.Precision.HIGHEST)
    h = jnp.maximum(h + b1_ref[...], 0.0)
    o_ref[...] = jnp.sum(h * w2_ref[...], axis=1, keepdims=True) + b2_ref[...]


def _mlp_tc(ue, ie, W1, b1, W2, b2):
    b1r = b1.reshape(1, H)
    w2r = W2.reshape(1, H)
    b2s = b2.reshape(1, 1)

    BLK = 2048
    return pl.pallas_call(
        _mlp_body,
        grid=(B // BLK,),
        in_specs=[
            pl.BlockSpec((BLK, D), lambda i: (i, 0)),
            pl.BlockSpec((BLK, D), lambda i: (i, 0)),
            pl.BlockSpec((D, H), lambda i: (0, 0)),
            pl.BlockSpec((D, H), lambda i: (0, 0)),
            pl.BlockSpec((1, H), lambda i: (0, 0)),
            pl.BlockSpec((1, H), lambda i: (0, 0)),
            pl.BlockSpec((1, 1), lambda i: (0, 0)),
        ],
        out_specs=pl.BlockSpec((BLK, 1), lambda i: (i, 0)),
        out_shape=jax.ShapeDtypeStruct((B, 1), jnp.float32),
    )(ue, ie, W1[:D], W1[D:], b1r, w2r, b2s)


def kernel(user, item, user_table, item_table, W1, b1, W2, b2):
    ue, ie = _gather_sc(user_table, item_table, user, item)
    return _mlp_tc(ue, ie, W1, b1, W2, b2)


# speedup vs baseline: 1.0003x; 1.0003x over previous
import dataclasses
import functools

import jax
import jax.numpy as jnp
from jax import lax
from jax.experimental import pallas as pl
from jax.experimental.pallas import tpu as pltpu
from jax.experimental.pallas import tpu_sc as plsc

B = 16384
D = 32
H = 128
NC = 2
NS = 16
NW = NC * NS
BPW = B // NW


def _sc_compiler_params():
    cp = pltpu.CompilerParams(use_tc_tiling_on_sc=True)
    if "needs_layout_passes" in pltpu.CompilerParams.__dataclass_fields__:
        cp = dataclasses.replace(cp, needs_layout_passes=False)
    return cp


def _gather_sc(user_table, item_table, user, item):
    mesh = plsc.VectorSubcoreMesh(core_axis_name="c", subcore_axis_name="s")

    @functools.partial(
        pl.kernel,
        mesh=mesh,
        out_type=[
            jax.ShapeDtypeStruct((B, D), jnp.float32),
            jax.ShapeDtypeStruct((B, D), jnp.float32),
        ],
        scratch_types=[
            pltpu.VMEM((BPW,), jnp.int32),
            pltpu.VMEM((BPW,), jnp.int32),
            pltpu.SemaphoreType.DMA,
        ],
        compiler_params=_sc_compiler_params(),
    )
    def k(ut_hbm, it_hbm, u_hbm, i_hbm, ue_hbm, ie_hbm, uv, iv, sem):
        wid = lax.axis_index("s") * NC + lax.axis_index("c")
        base = wid * BPW
        pltpu.sync_copy(u_hbm.at[pl.ds(base, BPW)], uv)
        pltpu.sync_copy(i_hbm.at[pl.ds(base, BPW)], iv)
        lanes = lax.iota(jnp.int32, 16)

        @pl.loop(0, BPW, step=16)
        def _(c):
            uchunk = uv[pl.ds(c, 16)]
            ichunk = iv[pl.ds(c, 16)]
            for j in range(16):
                uidx = jnp.sum(jnp.where(lanes == j, uchunk, 0))
                iidx = jnp.sum(jnp.where(lanes == j, ichunk, 0))
                pltpu.async_copy(ut_hbm.at[uidx], ue_hbm.at[base + c + j], sem)
                pltpu.async_copy(it_hbm.at[iidx], ie_hbm.at[base + c + j], sem)

        pltpu.make_async_copy(ut_hbm.at[pl.ds(0, BPW)],
                              ue_hbm.at[pl.ds(base, BPW)], sem).wait()
        pltpu.make_async_copy(it_hbm.at[pl.ds(0, BPW)],
                              ie_hbm.at[pl.ds(base, BPW)], sem).wait()

    return k(user_table, item_table, user, item)


def _mlp_body(ue_ref, ie_ref, w1u_ref, w1i_ref, b1_ref, w2_ref, b2_ref, o_ref):
    h = jnp.dot(ue_ref[...], w1u_ref[...], preferred_element_type=jnp.float32,
                precision=lax.Precision.HIGHEST)
    h = h + jnp.dot(ie_ref[...], w1i_ref[...], preferred_element_type=jnp.float32,
                    precision=lax.Precision.HIGHEST)
    h = jnp.maximum(h + b1_ref[...], 0.0)
    o_ref[...] = jnp.sum(h * w2_ref[...], axis=1, keepdims=True) + b2_ref[...]


def _mlp_tc(ue, ie, W1, b1, W2, b2):
    b1r = b1.reshape(1, H)
    w2r = W2.reshape(1, H)
    b2s = b2.reshape(1, 1)

    BLK = 2048
    return pl.pallas_call(
        _mlp_body,
        grid=(B // BLK,),
        in_specs=[
            pl.BlockSpec((BLK, D), lambda i: (i, 0)),
            pl.BlockSpec((BLK, D), lambda i: (i, 0)),
            pl.BlockSpec((D, H), lambda i: (0, 0)),
            pl.BlockSpec((D, H), lambda i: (0, 0)),
            pl.BlockSpec((1, H), lambda i: (0, 0)),
            pl.BlockSpec((1, H), lambda i: (0, 0)),
            pl.BlockSpec((1, 1), lambda i: (0, 0)),
        ],
        out_specs=pl.BlockSpec((BLK, 1), lambda i: (i, 0)),
        out_shape=jax.ShapeDtypeStruct((B, 1), jnp.float32),
    )(ue, ie, W1[:D], W1[D:], b1r, w2r, b2s)


def kernel(user, item, user_table, item_table, W1, b1, W2, b2):
    ue, ie = _gather_sc(user_table, item_table, user, item)
    return _mlp_tc(ue, ie, W1, b1, W2, b2)


# speedup vs baseline: 1.7547x; 1.7541x over previous
import functools

import jax
import jax.numpy as jnp
from jax import lax
from jax.experimental import pallas as pl
from jax.experimental.pallas import tpu as pltpu
from jax.experimental.pallas import tpu_sc as plsc

B = 16384
D = 32
H = 128
V = 1000000
NC = 2
NS = 16
NW = NC * NS
BPW = B // NW
CHUNK = BPW // 2
CB = 4096
NBLK = -(-V // CB)


def _packmm_body(x_ref, w_ref, o_ref):
    xb = x_ref[...].astype(jnp.bfloat16)
    wb = w_ref[...].astype(jnp.bfloat16)
    o_ref[...] = lax.dot_general(
        xb, wb, (((0,), (0,)), ((), ())),
        preferred_element_type=jnp.float32)


def _packmm_tc(table_t, w_half):
    return pl.pallas_call(
        _packmm_body,
        grid=(NBLK,),
        in_specs=[
            pl.BlockSpec((D, CB), lambda i: (0, i)),
            pl.BlockSpec((D, H), lambda i: (0, 0)),
        ],
        out_specs=pl.BlockSpec((CB, H), lambda i: (i, 0)),
        out_shape=jax.ShapeDtypeStruct((V, H), jnp.float32),
    )(table_t, w_half)


def _gather_one(table, idx):
    mesh = plsc.VectorSubcoreMesh(core_axis_name="c", subcore_axis_name="s")

    @functools.partial(
        pl.kernel,
        mesh=mesh,
        out_type=jax.ShapeDtypeStruct((B, H), jnp.float32),
        scratch_types=[
            pltpu.VMEM((BPW,), jnp.int32),
            pltpu.VMEM((CHUNK, H), jnp.float32),
            pltpu.VMEM((CHUNK, H), jnp.float32),
            pltpu.SemaphoreType.DMA,
            pltpu.SemaphoreType.DMA,
            pltpu.SemaphoreType.DMA,
            pltpu.SemaphoreType.DMA,
        ],
    )
    def k(t_hbm, i_hbm, o_hbm, idx_v, buf0, buf1, gs0, gs1, ws0, ws1):
        wid = lax.axis_index("s") * NC + lax.axis_index("c")
        base = wid * BPW
        pltpu.sync_copy(i_hbm.at[pl.ds(base, BPW)], idx_v)
        g0 = pltpu.async_copy(t_hbm.at[idx_v.at[pl.ds(0, CHUNK)]], buf0, gs0)
        g1 = pltpu.async_copy(t_hbm.at[idx_v.at[pl.ds(CHUNK, CHUNK)]], buf1, gs1)
        g0.wait()
        w0 = pltpu.async_copy(buf0, o_hbm.at[pl.ds(base, CHUNK)], ws0)
        g1.wait()
        w1 = pltpu.async_copy(buf1, o_hbm.at[pl.ds(base + CHUNK, CHUNK)], ws1)
        w0.wait()
        w1.wait()

    return k(table, idx)


def _mlp_body(gu_ref, gi_ref, b1_ref, w2_ref, b2_ref, o_ref):
    h = jnp.maximum(gu_ref[...] + gi_ref[...] + b1_ref[...], 0.0)
    o_ref[...] = jnp.sum(h * w2_ref[...], axis=1, keepdims=True) + b2_ref[...]


def _mlp_tc(gu, gi, b1, W2, b2):
    b1r = b1.reshape(1, H)
    w2r = W2.reshape(1, H)
    b2s = b2.reshape(1, 1)

    BLK = 2048
    return pl.pallas_call(
        _mlp_body,
        grid=(B // BLK,),
        in_specs=[
            pl.BlockSpec((BLK, H), lambda i: (i, 0)),
            pl.BlockSpec((BLK, H), lambda i: (i, 0)),
            pl.BlockSpec((1, H), lambda i: (0, 0)),
            pl.BlockSpec((1, H), lambda i: (0, 0)),
            pl.BlockSpec((1, 1), lambda i: (0, 0)),
        ],
        out_specs=pl.BlockSpec((BLK, 1), lambda i: (i, 0)),
        out_shape=jax.ShapeDtypeStruct((B, 1), jnp.float32),
    )(gu, gi, b1r, w2r, b2s)


def kernel(user, item, user_table, item_table, W1, b1, W2, b2):
    tu = _packmm_tc(user_table.T, W1[:D])
    gu = _gather_one(tu, user)
    ti = _packmm_tc(item_table.T, W1[D:])
    gi = _gather_one(ti, item)
    return _mlp_tc(gu, gi, b1, W2, b2)


# speedup vs baseline: 2.2845x; 1.3019x over previous
import functools

import jax
import jax.numpy as jnp
from jax import lax
from jax.experimental import pallas as pl
from jax.experimental.pallas import tpu as pltpu
from jax.experimental.pallas import tpu_sc as plsc

B = 16384
D = 32
H = 128
V = 1000000
NC = 2
NS = 16
NW = NC * NS
BPW = B // NW
CHUNK = BPW // 2
CB = 8192
NBLK = -(-V // CB)


def _packmm_body(x_ref, w_ref, o_ref):
    xb = x_ref[...].astype(jnp.bfloat16)
    wb = w_ref[...].astype(jnp.bfloat16)
    o_ref[...] = lax.dot_general(
        xb, wb, (((0,), (0,)), ((), ())),
        preferred_element_type=jnp.float32)


def _packmm_tc(table_t, w_half):
    return pl.pallas_call(
        _packmm_body,
        grid=(NBLK,),
        in_specs=[
            pl.BlockSpec((D, CB), lambda i: (0, i)),
            pl.BlockSpec((D, H), lambda i: (0, 0)),
        ],
        out_specs=pl.BlockSpec((CB, H), lambda i: (i, 0)),
        out_shape=jax.ShapeDtypeStruct((V, H), jnp.float32),
    )(table_t, w_half)


def _gather_one(table, idx):
    mesh = plsc.VectorSubcoreMesh(core_axis_name="c", subcore_axis_name="s")

    @functools.partial(
        pl.kernel,
        mesh=mesh,
        out_type=jax.ShapeDtypeStruct((B, H), jnp.float32),
        scratch_types=[
            pltpu.VMEM((BPW,), jnp.int32),
            pltpu.VMEM((CHUNK, H), jnp.float32),
            pltpu.VMEM((CHUNK, H), jnp.float32),
            pltpu.SemaphoreType.DMA,
            pltpu.SemaphoreType.DMA,
            pltpu.SemaphoreType.DMA,
            pltpu.SemaphoreType.DMA,
        ],
    )
    def k(t_hbm, i_hbm, o_hbm, idx_v, buf0, buf1, gs0, gs1, ws0, ws1):
        wid = lax.axis_index("s") * NC + lax.axis_index("c")
        base = wid * BPW
        pltpu.sync_copy(i_hbm.at[pl.ds(base, BPW)], idx_v)
        g0 = pltpu.async_copy(t_hbm.at[idx_v.at[pl.ds(0, CHUNK)]], buf0, gs0)
        g1 = pltpu.async_copy(t_hbm.at[idx_v.at[pl.ds(CHUNK, CHUNK)]], buf1, gs1)
        g0.wait()
        w0 = pltpu.async_copy(buf0, o_hbm.at[pl.ds(base, CHUNK)], ws0)
        g1.wait()
        w1 = pltpu.async_copy(buf1, o_hbm.at[pl.ds(base + CHUNK, CHUNK)], ws1)
        w0.wait()
        w1.wait()

    return k(table, idx)


def _mlp_body(gu_ref, gi_ref, b1_ref, w2_ref, b2_ref, o_ref):
    h = jnp.maximum(gu_ref[...] + gi_ref[...] + b1_ref[...], 0.0)
    o_ref[...] = jnp.sum(h * w2_ref[...], axis=1, keepdims=True) + b2_ref[...]


def _mlp_tc(gu, gi, b1, W2, b2):
    b1r = b1.reshape(1, H)
    w2r = W2.reshape(1, H)
    b2s = b2.reshape(1, 1)

    BLK = 2048
    return pl.pallas_call(
        _mlp_body,
        grid=(B // BLK,),
        in_specs=[
            pl.BlockSpec((BLK, H), lambda i: (i, 0)),
            pl.BlockSpec((BLK, H), lambda i: (i, 0)),
            pl.BlockSpec((1, H), lambda i: (0, 0)),
            pl.BlockSpec((1, H), lambda i: (0, 0)),
            pl.BlockSpec((1, 1), lambda i: (0, 0)),
        ],
        out_specs=pl.BlockSpec((BLK, 1), lambda i: (i, 0)),
        out_shape=jax.ShapeDtypeStruct((B, 1), jnp.float32),
    )(gu, gi, b1r, w2r, b2s)


def kernel(user, item, user_table, item_table, W1, b1, W2, b2):
    tu = _packmm_tc(user_table.T, W1[:D])
    gu = _gather_one(tu, user)
    ti = _packmm_tc(item_table.T, W1[D:])
    gi = _gather_one(ti, item)
    return _mlp_tc(gu, gi, b1, W2, b2)
